# Initial kernel scaffold; baseline (speedup 1.0000x reference)
import jax
import jax.numpy as jnp
from jax.experimental import pallas as pl


def kernel(x, pos_emb):
    raise NotImplementedError("write your pallas kernel here")


# speedup vs baseline: 3.5509x; 3.5509x over previous
import jax
import jax.numpy as jnp
from jax.experimental import pallas as pl


def _add_body(x_ref, p_ref, o_ref):
    o_ref[...] = x_ref[...] + p_ref[...][None, :, :]


def kernel(x, pos_emb):
    batch, seq_len, d_model = x.shape
    s_blk = 256
    grid = (seq_len // s_blk,)
    return pl.pallas_call(
        _add_body,
        grid=grid,
        in_specs=[
            pl.BlockSpec((batch, s_blk, d_model), lambda i: (0, i, 0)),
            pl.BlockSpec((s_blk, d_model), lambda i: (i, 0)),
        ],
        out_specs=pl.BlockSpec((batch, s_blk, d_model), lambda i: (0, i, 0)),
        out_shape=jax.ShapeDtypeStruct((batch, seq_len, d_model), x.dtype),
    )(x, pos_emb[:seq_len])


# speedup vs baseline: 3.6319x; 1.0228x over previous
import jax
import jax.numpy as jnp
from jax.experimental import pallas as pl


def _add_body(x_ref, p_ref, o_ref):
    o_ref[...] = x_ref[...] + p_ref[...][None, :, :]


def kernel(x, pos_emb):
    batch, seq_len, d_model = x.shape
    s_blk = 512
    grid = (seq_len // s_blk,)
    return pl.pallas_call(
        _add_body,
        grid=grid,
        in_specs=[
            pl.BlockSpec((batch, s_blk, d_model), lambda i: (0, i, 0)),
            pl.BlockSpec((s_blk, d_model), lambda i: (i, 0)),
        ],
        out_specs=pl.BlockSpec((batch, s_blk, d_model), lambda i: (0, i, 0)),
        out_shape=jax.ShapeDtypeStruct((batch, seq_len, d_model), x.dtype),
    )(x, pos_emb[:seq_len])


# speedup vs baseline: 3.6510x; 1.0053x over previous
import jax
import jax.numpy as jnp
from jax.experimental import pallas as pl


def _add_body(x_ref, p_ref, o_ref):
    o_ref[...] = x_ref[...] + p_ref[...][None, :, :]


def kernel(x, pos_emb):
    batch, seq_len, d_model = x.shape
    s_blk = 1024
    grid = (seq_len // s_blk,)
    return pl.pallas_call(
        _add_body,
        grid=grid,
        in_specs=[
            pl.BlockSpec((batch, s_blk, d_model), lambda i: (0, i, 0)),
            pl.BlockSpec((s_blk, d_model), lambda i: (i, 0)),
        ],
        out_specs=pl.BlockSpec((batch, s_blk, d_model), lambda i: (0, i, 0)),
        out_shape=jax.ShapeDtypeStruct((batch, seq_len, d_model), x.dtype),
    )(x, pos_emb[:seq_len])
